# baseline (speedup 1.0000x reference)
import functools

import jax
import jax.numpy as jnp
from jax import lax
from jax.experimental import pallas as pl
from jax.experimental.pallas import tpu as pltpu
from jax.experimental.pallas import tpu_sc as plsc

PREFIX_LEN = 5
N_CLS_CTX = 4
SUFFIX_LEN = 68
SEQ = PREFIX_LEN + N_CLS_CTX + SUFFIX_LEN
SEQ_ALIGNED = 72
D = 512

_SC_NUM_CORES = 2
_SC_NUM_SUBCORES = 16
_NW = _SC_NUM_CORES * _SC_NUM_SUBCORES


def _sc_gather(table, idx):
    row_shape = table.shape[1:]
    b = idx.shape[0]
    b_per_w = b // _NW
    mesh = plsc.VectorSubcoreMesh(core_axis_name="c", subcore_axis_name="s")

    @functools.partial(
        pl.kernel,
        mesh=mesh,
        out_type=jax.ShapeDtypeStruct((b,) + row_shape, table.dtype),
        scratch_types=[
            pltpu.VMEM((b_per_w,), jnp.int32),
            pltpu.VMEM((b_per_w,) + row_shape, table.dtype),
            pltpu.SemaphoreType.DMA,
        ],
    )
    def k(table_hbm, idx_hbm, out_hbm, idx_v, rows_v, sem):
        wid = lax.axis_index("s") * _SC_NUM_CORES + lax.axis_index("c")
        base = wid * b_per_w
        pltpu.sync_copy(idx_hbm.at[pl.ds(base, b_per_w)], idx_v)
        pltpu.async_copy(table_hbm.at[idx_v], rows_v, sem).wait()
        pltpu.sync_copy(rows_v, out_hbm.at[pl.ds(base, b_per_w)])

    return k(table, idx)


def _assemble_body(cls_ref, pre_ref, suf_ref, out_hbm, tmpl_ref, buf_ref, sem):
    i = pl.program_id(0)
    gb = buf_ref.shape[0]

    @pl.when(i == 0)
    def _():
        tmpl_ref[0:PREFIX_LEN, :] = pre_ref[0]
        tmpl_ref[PREFIX_LEN:PREFIX_LEN + N_CLS_CTX, :] = jnp.zeros(
            (N_CLS_CTX, D), buf_ref.dtype)
        tmpl_ref[PREFIX_LEN + N_CLS_CTX:, :] = suf_ref[0]

    buf_ref[...] = jnp.broadcast_to(tmpl_ref[...][None], (gb, SEQ, D))
    buf_ref[:, PREFIX_LEN:PREFIX_LEN + N_CLS_CTX, :] = cls_ref[...]

    cp_body = pltpu.make_async_copy(
        buf_ref.at[:, pl.ds(0, SEQ_ALIGNED), :],
        out_hbm.at[pl.ds(i * gb, gb), pl.ds(0, SEQ_ALIGNED), :],
        sem.at[0])
    cp_tail = pltpu.make_async_copy(
        buf_ref.at[:, pl.ds(SEQ_ALIGNED, SEQ - SEQ_ALIGNED), :],
        out_hbm.at[pl.ds(i * gb, gb), pl.ds(SEQ_ALIGNED, SEQ - SEQ_ALIGNED), :],
        sem.at[1])
    cp_body.start()
    cp_tail.start()
    cp_body.wait()
    cp_tail.wait()


def _tc_assemble(cls_g, token_prefix, token_suffix, gb=64):
    b = cls_g.shape[0]
    grid = (b // gb,)
    return pl.pallas_call(
        _assemble_body,
        grid=grid,
        in_specs=[
            pl.BlockSpec((gb, N_CLS_CTX, D), lambda i: (i, 0, 0)),
            pl.BlockSpec((1, PREFIX_LEN, D), lambda i: (0, 0, 0)),
            pl.BlockSpec((1, SUFFIX_LEN, D), lambda i: (0, 0, 0)),
        ],
        out_specs=pl.BlockSpec(memory_space=pl.ANY),
        out_shape=jax.ShapeDtypeStruct((b, SEQ, D), cls_g.dtype),
        scratch_shapes=[
            pltpu.VMEM((SEQ, D), cls_g.dtype),
            pltpu.VMEM((gb, SEQ, D), cls_g.dtype),
            pltpu.SemaphoreType.DMA((2,)),
        ],
    )(cls_g, token_prefix, token_suffix)


def kernel(label, cls_ctx, token_prefix, token_suffix):
    cls_g = _sc_gather(cls_ctx, label)
    return _tc_assemble(cls_g, token_prefix, token_suffix)


# speedup vs baseline: 2.6395x; 2.6395x over previous
import functools

import jax
import jax.numpy as jnp
from jax import lax
from jax.experimental import pallas as pl
from jax.experimental.pallas import tpu as pltpu
from jax.experimental.pallas import tpu_sc as plsc

PREFIX_LEN = 5
N_CLS_CTX = 4
SUFFIX_LEN = 68
SEQ = PREFIX_LEN + N_CLS_CTX + SUFFIX_LEN
D = 512

_SC_NUM_CORES = 2
_SC_NUM_SUBCORES = 16
_NW = _SC_NUM_CORES * _SC_NUM_SUBCORES


def _sc_gather_slab(table, idx):
    b = idx.shape[0]
    b_per_w = b // _NW
    mesh = plsc.VectorSubcoreMesh(core_axis_name="c", subcore_axis_name="s")

    @functools.partial(
        pl.kernel,
        mesh=mesh,
        out_type=jax.ShapeDtypeStruct((N_CLS_CTX, b, D), table.dtype),
        scratch_types=[
            pltpu.VMEM((b_per_w,), jnp.int32),
            pltpu.VMEM((b_per_w, N_CLS_CTX, D), table.dtype),
            pltpu.SemaphoreType.DMA,
        ],
    )
    def k(table_hbm, idx_hbm, out_hbm, idx_v, rows_v, sem):
        wid = lax.axis_index("s") * _SC_NUM_CORES + lax.axis_index("c")
        base = wid * b_per_w
        pltpu.sync_copy(idx_hbm.at[pl.ds(base, b_per_w)], idx_v)
        pltpu.async_copy(table_hbm.at[idx_v], rows_v, sem).wait()
        for kk in range(N_CLS_CTX):
            pltpu.sync_copy(rows_v.at[:, kk, :],
                            out_hbm.at[kk, pl.ds(base, b_per_w), :])

    return k(table, idx)


def _assemble_body(cls_ref, pre_ref, suf_ref, out_ref):
    bb = out_ref.shape[1]
    for s in range(PREFIX_LEN):
        out_ref[s] = jnp.broadcast_to(pre_ref[0, s][None], (bb, D))
    for s in range(N_CLS_CTX):
        out_ref[PREFIX_LEN + s] = cls_ref[s]
    for s in range(SUFFIX_LEN):
        out_ref[PREFIX_LEN + N_CLS_CTX + s] = jnp.broadcast_to(
            suf_ref[0, s][None], (bb, D))


def _tc_assemble_slab(cls_slab, token_prefix, token_suffix, bb=64):
    b = cls_slab.shape[1]
    grid = (b // bb,)
    return pl.pallas_call(
        _assemble_body,
        grid=grid,
        in_specs=[
            pl.BlockSpec((N_CLS_CTX, bb, D), lambda i: (0, i, 0)),
            pl.BlockSpec((1, PREFIX_LEN, D), lambda i: (0, 0, 0)),
            pl.BlockSpec((1, SUFFIX_LEN, D), lambda i: (0, 0, 0)),
        ],
        out_specs=pl.BlockSpec((SEQ, bb, D), lambda i: (0, i, 0)),
        out_shape=jax.ShapeDtypeStruct((SEQ, b, D), cls_slab.dtype),
    )(cls_slab, token_prefix, token_suffix)


def kernel(label, cls_ctx, token_prefix, token_suffix):
    cls_slab = _sc_gather_slab(cls_ctx, label)
    out_t = _tc_assemble_slab(cls_slab, token_prefix, token_suffix)
    return jnp.transpose(out_t, (1, 0, 2))
